# baseline (speedup 1.0000x reference)
import functools

import jax
import jax.numpy as jnp
from jax import lax
from jax.experimental import pallas as pl
from jax.experimental.pallas import tpu as pltpu

B, S, D = 2, 2048, 1024
E = 8
K = 2
FF = 4096
T = B * S
NP = T * K
TM = 512
NT = (NP + E * (TM - 1) + TM - 1) // TM
P = NT * TM
BT = 512


def _gate_body(x_ref, gw_ref, gb_ref, ef_ref, rf_ref, w_ref, cnt_ref,
               ends_ref, offs_ref, carry_ref):
    pid = pl.program_id(0)

    @pl.when(pid == 0)
    def _():
        carry_ref[...] = jnp.zeros_like(carry_ref)

    x = x_ref[...]
    logits = jnp.dot(x, gw_ref[...], preferred_element_type=jnp.float32)
    logits = logits + gb_ref[...]
    eids = lax.broadcasted_iota(jnp.int32, (BT, E), 1)
    m1 = jnp.max(logits, axis=1, keepdims=True)
    a1 = jnp.min(jnp.where(logits == m1, eids, E), axis=1, keepdims=True)
    masked = jnp.where(eids == a1, -1e30, logits)
    m2 = jnp.max(masked, axis=1, keepdims=True)
    a2 = jnp.min(jnp.where(masked == m2, eids, E), axis=1, keepdims=True)
    t = jnp.exp(m2 - m1)
    w1 = 1.0 / (1.0 + t)
    w2 = t * w1

    oh1 = (eids == a1).astype(jnp.float32)
    oh2 = (eids == a2).astype(jnp.float32)
    both = oh1 + oh2
    ti = lax.broadcasted_iota(jnp.int32, (BT, BT), 0)
    tj = lax.broadcasted_iota(jnp.int32, (BT, BT), 1)
    tril = (tj < ti).astype(jnp.float32)
    csum = jnp.dot(tril, both, preferred_element_type=jnp.float32)
    base = csum + carry_ref[...]
    r1 = jnp.sum(oh1 * base, axis=1, keepdims=True)
    r2 = jnp.sum(oh2 * (base + oh1), axis=1, keepdims=True)
    carry_new = carry_ref[...] + jnp.sum(both, axis=0, keepdims=True)
    carry_ref[...] = carry_new

    ef_ref[...] = jnp.concatenate([a1, a2], axis=1)
    rf_ref[...] = jnp.concatenate([r1, r2], axis=1).astype(jnp.int32)
    w_ref[...] = jnp.concatenate([w1, w2], axis=1)
    cnt_ref[...] = carry_new.astype(jnp.int32)
    cpv = ((carry_new.astype(jnp.int32) + (TM - 1)) & ~(TM - 1)).astype(jnp.float32)
    li = lax.broadcasted_iota(jnp.int32, (E, E), 0)
    lj = lax.broadcasted_iota(jnp.int32, (E, E), 1)
    lt8 = (li <= lj).astype(jnp.float32)
    endsf = jnp.dot(cpv, lt8, preferred_element_type=jnp.float32)
    ends_ref[...] = endsf.astype(jnp.int32)
    offs_ref[...] = (endsf - cpv).astype(jnp.int32)


def _gate_call(x, gate_W, gate_b):
    grid = (T // BT,)
    return pl.pallas_call(
        _gate_body,
        grid=grid,
        in_specs=[
            pl.BlockSpec((BT, D), lambda i: (i, 0)),
            pl.BlockSpec((D, E), lambda i: (0, 0)),
            pl.BlockSpec((E,), lambda i: (0,)),
        ],
        out_specs=[
            pl.BlockSpec((BT, K), lambda i: (i, 0)),
            pl.BlockSpec((BT, K), lambda i: (i, 0)),
            pl.BlockSpec((BT, K), lambda i: (i, 0)),
            pl.BlockSpec((1, E), lambda i: (0, 0)),
            pl.BlockSpec((1, E), lambda i: (0, 0)),
            pl.BlockSpec((1, E), lambda i: (0, 0)),
        ],
        out_shape=[
            jax.ShapeDtypeStruct((T, K), jnp.int32),
            jax.ShapeDtypeStruct((T, K), jnp.int32),
            jax.ShapeDtypeStruct((T, K), jnp.float32),
            jax.ShapeDtypeStruct((1, E), jnp.int32),
            jax.ShapeDtypeStruct((1, E), jnp.int32),
            jax.ShapeDtypeStruct((1, E), jnp.int32),
        ],
        scratch_shapes=[pltpu.VMEM((1, E), jnp.float32)],
    )(x, gate_W, gate_b)


NSUB = 4
FC = FF // NSUB


def _wcopy(w1_hbm, w2_hbm, w1s, w2s, sem1, sem2, e, slot):
    c1 = pltpu.make_async_copy(w1_hbm.at[e], w1s.at[slot], sem1.at[slot])
    c2 = pltpu.make_async_copy(w2_hbm.at[e], w2s.at[slot], sem2.at[slot])
    return c1, c2


def _ffn_body(te_ref, chg_ref, slot_ref, nxt_ref, hn_ref,
              xd_ref, w1_hbm, b1_ref, w2_hbm, b2_ref, y_ref,
              w1s, w2s, sem1, sem2):
    i = pl.program_id(0)
    e = te_ref[i]
    sl = slot_ref[i]

    @pl.when(i == 0)
    def _():
        c1, c2 = _wcopy(w1_hbm, w2_hbm, w1s, w2s, sem1, sem2, e, sl)
        c1.start()
        c2.start()

    @pl.when(chg_ref[i] == 1)
    def _():
        c1, c2 = _wcopy(w1_hbm, w2_hbm, w1s, w2s, sem1, sem2, e, sl)
        c1.wait()
        c2.wait()

        @pl.when(hn_ref[i] == 1)
        def _():
            p1, p2 = _wcopy(w1_hbm, w2_hbm, w1s, w2s, sem1, sem2,
                            nxt_ref[i], 1 - sl)
            p1.start()
            p2.start()

    x = xd_ref[...].astype(jnp.bfloat16)
    acc = None
    for s in range(NSUB):
        h = jnp.dot(x, w1s[sl, :, s * FC:(s + 1) * FC],
                    preferred_element_type=jnp.float32) + b1_ref[0, 0, s * FC:(s + 1) * FC]
        h = jax.nn.gelu(h).astype(jnp.bfloat16)
        part = jnp.dot(h, w2s[sl, s * FC:(s + 1) * FC, :],
                       preferred_element_type=jnp.float32)
        acc = part if acc is None else acc + part
    y_ref[...] = acc + b2_ref[0]


def _ffn_call(te, chg, slot, nxt, hn, xd, W1, b1, W2, b2):
    grid_spec = pltpu.PrefetchScalarGridSpec(
        num_scalar_prefetch=5,
        grid=(NT,),
        in_specs=[
            pl.BlockSpec((TM, D), lambda i, *_: (i, 0)),
            pl.BlockSpec(memory_space=pl.ANY),
            pl.BlockSpec((1, 1, FF), lambda i, te, *_: (te[i], 0, 0)),
            pl.BlockSpec(memory_space=pl.ANY),
            pl.BlockSpec((1, 1, D), lambda i, te, *_: (te[i], 0, 0)),
        ],
        out_specs=pl.BlockSpec((TM, D), lambda i, *_: (i, 0)),
        scratch_shapes=[
            pltpu.VMEM((2, D, FF), jnp.bfloat16),
            pltpu.VMEM((2, FF, D), jnp.bfloat16),
            pltpu.SemaphoreType.DMA((2,)),
            pltpu.SemaphoreType.DMA((2,)),
        ],
    )
    return pl.pallas_call(
        _ffn_body,
        grid_spec=grid_spec,
        out_shape=jax.ShapeDtypeStruct((P, D), jnp.float32),
    )(te, chg, slot, nxt, hn, xd, W1.astype(jnp.bfloat16), b1.reshape(E, 1, FF),
      W2.astype(jnp.bfloat16), b2.reshape(E, 1, D))


from jax.experimental.pallas import tpu_sc as plsc

NC, NS, L = 2, 16, 16
NW = NC * NS
TPW = T // NW
NTE = 48
_sc_mesh = plsc.VectorSubcoreMesh(core_axis_name="c", subcore_axis_name="s")


_GDN = lax.GatherDimensionNumbers(offset_dims=(), collapsed_slice_dims=(0,),
                                 start_index_map=(0,))


def _vtake(vec, idx):
    return lax.gather(vec, idx[:, None], _GDN, (1,),
                      mode=lax.GatherScatterMode.PROMISE_IN_BOUNDS)


def _dispatch_body(offs_hbm, e0_hbm, e1_hbm, r0_hbm, r1_hbm, x_hbm,
                   dst0_hbm, dst1_hbm, xd_hbm,
                   offs_v, e_v, r_v, d0_v, d1_v,
                   rows_v, sem):
    wid = lax.axis_index("s") * NC + lax.axis_index("c")
    tbase = wid * TPW

    pltpu.sync_copy(offs_hbm, offs_v)

    pltpu.sync_copy(e0_hbm.at[pl.ds(tbase, TPW)], e_v)
    pltpu.sync_copy(r0_hbm.at[pl.ds(tbase, TPW)], r_v)
    for cix in range(TPW // L):
        ech = e_v[pl.ds(cix * L, L)]
        rch = r_v[pl.ds(cix * L, L)]
        och = rch
        for e in range(E):
            och = och + jnp.where(ech == e, offs_v[e], 0)
        d0_v[cix // 2, pl.ds((cix % 2) * L, L)] = och
    pltpu.sync_copy(e1_hbm.at[pl.ds(tbase, TPW)], e_v)
    pltpu.sync_copy(r1_hbm.at[pl.ds(tbase, TPW)], r_v)
    for cix in range(TPW // L):
        ech = e_v[pl.ds(cix * L, L)]
        rch = r_v[pl.ds(cix * L, L)]
        och = rch
        for e in range(E):
            och = och + jnp.where(ech == e, offs_v[e], 0)
        d1_v[cix // 2, pl.ds((cix % 2) * L, L)] = och
    for g in range(4):
        pltpu.sync_copy(d0_v.at[g], dst0_hbm.at[pl.ds(tbase + g * 32, 32)])
        pltpu.sync_copy(d1_v.at[g], dst1_hbm.at[pl.ds(tbase + g * 32, 32)])

    for g in range(4):
        pltpu.sync_copy(x_hbm.at[pl.ds(tbase + g * 32, 32)], rows_v)
        a = pltpu.async_copy(rows_v, xd_hbm.at[d0_v.at[g]], sem)
        b = pltpu.async_copy(rows_v, xd_hbm.at[d1_v.at[g]], sem)
        a.wait()
        b.wait()


def _dispatch_sc(offsb, e0, e1, r0, r1, x):
    f = pl.kernel(
        _dispatch_body,
        out_type=[
            jax.ShapeDtypeStruct((T,), jnp.int32),
            jax.ShapeDtypeStruct((T,), jnp.int32),
            jax.ShapeDtypeStruct((P, D), jnp.float32),
        ],
        mesh=_sc_mesh,
        scratch_types=[
            pltpu.VMEM((E, L), jnp.int32),
            pltpu.VMEM((TPW,), jnp.int32),
            pltpu.VMEM((TPW,), jnp.int32),
            pltpu.VMEM((4, 32), jnp.int32),
            pltpu.VMEM((4, 32), jnp.int32),
            pltpu.VMEM((32, D), jnp.float32),
            pltpu.SemaphoreType.DMA,
        ],
    )
    return f(offsb, e0, e1, r0, r1, x)


def _combine_body(y_hbm, dst0_hbm, dst1_hbm, w0_hbm, w1_hbm, out_hbm,
                  d0_v, d1_v, w0_v, w1_v, rows0_v, rows1_v, out_v, sem):
    wid = lax.axis_index("s") * NC + lax.axis_index("c")
    tbase = wid * TPW
    pltpu.sync_copy(dst0_hbm.at[pl.ds(tbase, TPW)], d0_v)
    pltpu.sync_copy(dst1_hbm.at[pl.ds(tbase, TPW)], d1_v)
    pltpu.sync_copy(w0_hbm.at[pl.ds(tbase, TPW)], w0_v)
    pltpu.sync_copy(w1_hbm.at[pl.ds(tbase, TPW)], w1_v)
    for g in range(TPW // L):
        a = pltpu.async_copy(y_hbm.at[d0_v.at[pl.ds(g * L, L)]], rows0_v, sem)
        b = pltpu.async_copy(y_hbm.at[d1_v.at[pl.ds(g * L, L)]], rows1_v, sem)
        a.wait()
        b.wait()
        for t in range(L):
            w0b = w0_v[g * L + t]
            w1b = w1_v[g * L + t]

            def body(j, _):
                for u in range(4):
                    sl = pl.ds((j * 4 + u) * L, L)
                    out_v[t, sl] = w0b * rows0_v[t, sl] + w1b * rows1_v[t, sl]
                return 0

            lax.fori_loop(0, D // (4 * L), body, 0)
        pltpu.sync_copy(out_v, out_hbm.at[pl.ds(tbase + g * L, L)])


def _combine_sc(y, dst0, dst1, w0, w1):
    f = pl.kernel(
        _combine_body,
        out_type=jax.ShapeDtypeStruct((T, D), jnp.float32),
        mesh=_sc_mesh,
        scratch_types=[
            pltpu.VMEM((TPW,), jnp.int32),
            pltpu.VMEM((TPW,), jnp.int32),
            pltpu.VMEM((TPW, L), jnp.float32),
            pltpu.VMEM((TPW, L), jnp.float32),
            pltpu.VMEM((L, D), jnp.float32),
            pltpu.VMEM((L, D), jnp.float32),
            pltpu.VMEM((L, D), jnp.float32),
            pltpu.SemaphoreType.DMA,
        ],
    )
    return f(y, dst0, dst1, w0, w1)


def kernel(inputs, gate_W, gate_b, W1, b1, W2, b2):
    x = inputs.reshape(T, D)
    ef, rf, w, counts, ends8, offs8 = _gate_call(x, gate_W, gate_b)
    offsb = jnp.broadcast_to(offs8.reshape(E, 1), (E, L))

    e0 = ef[:, 0].astype(jnp.int32)
    e1 = ef[:, 1].astype(jnp.int32)
    r0 = rf[:, 0].astype(jnp.int32)
    r1 = rf[:, 1].astype(jnp.int32)

    dst0, dst1, xd = _dispatch_sc(offsb, e0, e1, r0, r1, x)
    te = jnp.clip(
        jnp.searchsorted(ends8.reshape(E), jnp.arange(NT, dtype=jnp.int32) * TM,
                         side="right"),
        0, E - 1).astype(jnp.int32)

    chg = jnp.concatenate([jnp.ones((1,), jnp.int32),
                           (te[1:] != te[:-1]).astype(jnp.int32)])
    visit = jnp.cumsum(chg) - 1
    slot = (visit % 2).astype(jnp.int32)
    ev = jnp.zeros((NT,), jnp.int32).at[visit].set(te)
    nvisit = visit[-1] + 1
    hn = (visit < nvisit - 1).astype(jnp.int32)
    nxt = ev[jnp.minimum(visit + 1, NT - 1)].astype(jnp.int32)

    y = _ffn_call(te, chg, slot, nxt, hn, xd, W1, b1, W2, b2)

    out = _combine_sc(y, dst0, dst1,
                      jnp.broadcast_to(w[:, 0:1], (T, L)),
                      jnp.broadcast_to(w[:, 1:2], (T, L)))
    return out.reshape(B, S, D)


# speedup vs baseline: 1.0267x; 1.0267x over previous
import functools

import jax
import jax.numpy as jnp
from jax import lax
from jax.experimental import pallas as pl
from jax.experimental.pallas import tpu as pltpu

B, S, D = 2, 2048, 1024
E = 8
K = 2
FF = 4096
T = B * S
NP = T * K
TM = 256
NT = (NP + E * (TM - 1) + TM - 1) // TM
P = NT * TM
BT = 512


def _gate_body(x_ref, gw_ref, gb_ref, ef_ref, rf_ref, w_ref, cnt_ref,
               ends_ref, offs_ref, carry_ref):
    pid = pl.program_id(0)

    @pl.when(pid == 0)
    def _():
        carry_ref[...] = jnp.zeros_like(carry_ref)

    x = x_ref[...]
    logits = jnp.dot(x, gw_ref[...], preferred_element_type=jnp.float32)
    logits = logits + gb_ref[...]
    eids = lax.broadcasted_iota(jnp.int32, (BT, E), 1)
    m1 = jnp.max(logits, axis=1, keepdims=True)
    a1 = jnp.min(jnp.where(logits == m1, eids, E), axis=1, keepdims=True)
    masked = jnp.where(eids == a1, -1e30, logits)
    m2 = jnp.max(masked, axis=1, keepdims=True)
    a2 = jnp.min(jnp.where(masked == m2, eids, E), axis=1, keepdims=True)
    t = jnp.exp(m2 - m1)
    w1 = 1.0 / (1.0 + t)
    w2 = t * w1

    oh1 = (eids == a1).astype(jnp.float32)
    oh2 = (eids == a2).astype(jnp.float32)
    both = oh1 + oh2
    ti = lax.broadcasted_iota(jnp.int32, (BT, BT), 0)
    tj = lax.broadcasted_iota(jnp.int32, (BT, BT), 1)
    tril = (tj < ti).astype(jnp.float32)
    csum = jnp.dot(tril, both, preferred_element_type=jnp.float32)
    base = csum + carry_ref[...]
    r1 = jnp.sum(oh1 * base, axis=1, keepdims=True)
    r2 = jnp.sum(oh2 * (base + oh1), axis=1, keepdims=True)
    carry_new = carry_ref[...] + jnp.sum(both, axis=0, keepdims=True)
    carry_ref[...] = carry_new

    ef_ref[...] = jnp.concatenate([a1, a2], axis=1)
    rf_ref[...] = jnp.concatenate([r1, r2], axis=1).astype(jnp.int32)
    w_ref[...] = jnp.concatenate([w1, w2], axis=1)
    cnt_ref[...] = carry_new.astype(jnp.int32)
    cpv = ((carry_new.astype(jnp.int32) + (TM - 1)) & ~(TM - 1)).astype(jnp.float32)
    li = lax.broadcasted_iota(jnp.int32, (E, E), 0)
    lj = lax.broadcasted_iota(jnp.int32, (E, E), 1)
    lt8 = (li <= lj).astype(jnp.float32)
    endsf = jnp.dot(cpv, lt8, preferred_element_type=jnp.float32)
    ends_ref[...] = endsf.astype(jnp.int32)
    offs_ref[...] = (endsf - cpv).astype(jnp.int32)


def _gate_call(x, gate_W, gate_b):
    grid = (T // BT,)
    return pl.pallas_call(
        _gate_body,
        grid=grid,
        in_specs=[
            pl.BlockSpec((BT, D), lambda i: (i, 0)),
            pl.BlockSpec((D, E), lambda i: (0, 0)),
            pl.BlockSpec((E,), lambda i: (0,)),
        ],
        out_specs=[
            pl.BlockSpec((BT, K), lambda i: (i, 0)),
            pl.BlockSpec((BT, K), lambda i: (i, 0)),
            pl.BlockSpec((BT, K), lambda i: (i, 0)),
            pl.BlockSpec((1, E), lambda i: (0, 0)),
            pl.BlockSpec((1, E), lambda i: (0, 0)),
            pl.BlockSpec((1, E), lambda i: (0, 0)),
        ],
        out_shape=[
            jax.ShapeDtypeStruct((T, K), jnp.int32),
            jax.ShapeDtypeStruct((T, K), jnp.int32),
            jax.ShapeDtypeStruct((T, K), jnp.float32),
            jax.ShapeDtypeStruct((1, E), jnp.int32),
            jax.ShapeDtypeStruct((1, E), jnp.int32),
            jax.ShapeDtypeStruct((1, E), jnp.int32),
        ],
        scratch_shapes=[pltpu.VMEM((1, E), jnp.float32)],
    )(x, gate_W, gate_b)


NSUB = 2
FC = FF // NSUB


def _wcopy(w1_hbm, w2_hbm, w1s, w2s, sem1, sem2, e, slot):
    c1 = pltpu.make_async_copy(w1_hbm.at[e], w1s.at[slot], sem1.at[slot])
    c2 = pltpu.make_async_copy(w2_hbm.at[e], w2s.at[slot], sem2.at[slot])
    return c1, c2


def _ffn_body(te_ref, chg_ref, slot_ref, nxt_ref, hn_ref,
              xd_ref, w1_hbm, b1_ref, w2_hbm, b2_ref, y_ref,
              w1s, w2s, sem1, sem2):
    i = pl.program_id(0)
    e = te_ref[i]
    sl = slot_ref[i]

    @pl.when(i == 0)
    def _():
        c1, c2 = _wcopy(w1_hbm, w2_hbm, w1s, w2s, sem1, sem2, e, sl)
        c1.start()
        c2.start()

    @pl.when(chg_ref[i] == 1)
    def _():
        c1, c2 = _wcopy(w1_hbm, w2_hbm, w1s, w2s, sem1, sem2, e, sl)
        c1.wait()
        c2.wait()

        @pl.when(hn_ref[i] == 1)
        def _():
            p1, p2 = _wcopy(w1_hbm, w2_hbm, w1s, w2s, sem1, sem2,
                            nxt_ref[i], 1 - sl)
            p1.start()
            p2.start()

    x = xd_ref[...].astype(jnp.bfloat16)
    acc = None
    for s in range(NSUB):
        h = jnp.dot(x, w1s[sl, :, s * FC:(s + 1) * FC],
                    preferred_element_type=jnp.float32) + b1_ref[0, 0, s * FC:(s + 1) * FC]
        h = jax.nn.gelu(h).astype(jnp.bfloat16)
        part = jnp.dot(h, w2s[sl, s * FC:(s + 1) * FC, :],
                       preferred_element_type=jnp.float32)
        acc = part if acc is None else acc + part
    y_ref[...] = acc + b2_ref[0]


def _ffn_call(te, chg, slot, nxt, hn, xd, W1, b1, W2, b2):
    grid_spec = pltpu.PrefetchScalarGridSpec(
        num_scalar_prefetch=5,
        grid=(NT,),
        in_specs=[
            pl.BlockSpec((TM, D), lambda i, *_: (i, 0)),
            pl.BlockSpec(memory_space=pl.ANY),
            pl.BlockSpec((1, 1, FF), lambda i, te, *_: (te[i], 0, 0)),
            pl.BlockSpec(memory_space=pl.ANY),
            pl.BlockSpec((1, 1, D), lambda i, te, *_: (te[i], 0, 0)),
        ],
        out_specs=pl.BlockSpec((TM, D), lambda i, *_: (i, 0)),
        scratch_shapes=[
            pltpu.VMEM((2, D, FF), jnp.bfloat16),
            pltpu.VMEM((2, FF, D), jnp.bfloat16),
            pltpu.SemaphoreType.DMA((2,)),
            pltpu.SemaphoreType.DMA((2,)),
        ],
    )
    return pl.pallas_call(
        _ffn_body,
        grid_spec=grid_spec,
        out_shape=jax.ShapeDtypeStruct((P, D), jnp.float32),
    )(te, chg, slot, nxt, hn, xd, W1.astype(jnp.bfloat16), b1.reshape(E, 1, FF),
      W2.astype(jnp.bfloat16), b2.reshape(E, 1, D))


from jax.experimental.pallas import tpu_sc as plsc

NC, NS, L = 2, 16, 16
NW = NC * NS
TPW = T // NW
NTE = 48
_sc_mesh = plsc.VectorSubcoreMesh(core_axis_name="c", subcore_axis_name="s")


_GDN = lax.GatherDimensionNumbers(offset_dims=(), collapsed_slice_dims=(0,),
                                 start_index_map=(0,))


def _vtake(vec, idx):
    return lax.gather(vec, idx[:, None], _GDN, (1,),
                      mode=lax.GatherScatterMode.PROMISE_IN_BOUNDS)


def _dispatch_body(offs_hbm, e0_hbm, e1_hbm, r0_hbm, r1_hbm, x_hbm,
                   dst0_hbm, dst1_hbm, xd_hbm,
                   offs_v, e_v, r_v, d0_v, d1_v,
                   rows_v, sem):
    wid = lax.axis_index("s") * NC + lax.axis_index("c")
    tbase = wid * TPW

    pltpu.sync_copy(offs_hbm, offs_v)

    pltpu.sync_copy(e0_hbm.at[pl.ds(tbase, TPW)], e_v)
    pltpu.sync_copy(r0_hbm.at[pl.ds(tbase, TPW)], r_v)
    for cix in range(TPW // L):
        ech = e_v[pl.ds(cix * L, L)]
        rch = r_v[pl.ds(cix * L, L)]
        och = rch
        for e in range(E):
            och = och + jnp.where(ech == e, offs_v[e], 0)
        d0_v[cix // 2, pl.ds((cix % 2) * L, L)] = och
    pltpu.sync_copy(e1_hbm.at[pl.ds(tbase, TPW)], e_v)
    pltpu.sync_copy(r1_hbm.at[pl.ds(tbase, TPW)], r_v)
    for cix in range(TPW // L):
        ech = e_v[pl.ds(cix * L, L)]
        rch = r_v[pl.ds(cix * L, L)]
        och = rch
        for e in range(E):
            och = och + jnp.where(ech == e, offs_v[e], 0)
        d1_v[cix // 2, pl.ds((cix % 2) * L, L)] = och
    for g in range(4):
        pltpu.sync_copy(d0_v.at[g], dst0_hbm.at[pl.ds(tbase + g * 32, 32)])
        pltpu.sync_copy(d1_v.at[g], dst1_hbm.at[pl.ds(tbase + g * 32, 32)])

    for g in range(4):
        pltpu.sync_copy(x_hbm.at[pl.ds(tbase + g * 32, 32)], rows_v)
        a = pltpu.async_copy(rows_v, xd_hbm.at[d0_v.at[g]], sem)
        b = pltpu.async_copy(rows_v, xd_hbm.at[d1_v.at[g]], sem)
        a.wait()
        b.wait()


def _dispatch_sc(offsb, e0, e1, r0, r1, x):
    f = pl.kernel(
        _dispatch_body,
        out_type=[
            jax.ShapeDtypeStruct((T,), jnp.int32),
            jax.ShapeDtypeStruct((T,), jnp.int32),
            jax.ShapeDtypeStruct((P, D), jnp.float32),
        ],
        mesh=_sc_mesh,
        scratch_types=[
            pltpu.VMEM((E, L), jnp.int32),
            pltpu.VMEM((TPW,), jnp.int32),
            pltpu.VMEM((TPW,), jnp.int32),
            pltpu.VMEM((4, 32), jnp.int32),
            pltpu.VMEM((4, 32), jnp.int32),
            pltpu.VMEM((32, D), jnp.float32),
            pltpu.SemaphoreType.DMA,
        ],
    )
    return f(offsb, e0, e1, r0, r1, x)


def _combine_body(y_hbm, dst0_hbm, dst1_hbm, w0_hbm, w1_hbm, out_hbm,
                  d0_v, d1_v, w0_v, w1_v, rows0_v, rows1_v, out_v, sem):
    wid = lax.axis_index("s") * NC + lax.axis_index("c")
    tbase = wid * TPW
    pltpu.sync_copy(dst0_hbm.at[pl.ds(tbase, TPW)], d0_v)
    pltpu.sync_copy(dst1_hbm.at[pl.ds(tbase, TPW)], d1_v)
    pltpu.sync_copy(w0_hbm.at[pl.ds(tbase, TPW)], w0_v)
    pltpu.sync_copy(w1_hbm.at[pl.ds(tbase, TPW)], w1_v)
    for g in range(TPW // L):
        a = pltpu.async_copy(y_hbm.at[d0_v.at[pl.ds(g * L, L)]], rows0_v, sem)
        b = pltpu.async_copy(y_hbm.at[d1_v.at[pl.ds(g * L, L)]], rows1_v, sem)
        a.wait()
        b.wait()
        for t in range(L):
            w0b = w0_v[g * L + t]
            w1b = w1_v[g * L + t]

            def body(j, _):
                for u in range(4):
                    sl = pl.ds((j * 4 + u) * L, L)
                    out_v[t, sl] = w0b * rows0_v[t, sl] + w1b * rows1_v[t, sl]
                return 0

            lax.fori_loop(0, D // (4 * L), body, 0)
        pltpu.sync_copy(out_v, out_hbm.at[pl.ds(tbase + g * L, L)])


def _combine_sc(y, dst0, dst1, w0, w1):
    f = pl.kernel(
        _combine_body,
        out_type=jax.ShapeDtypeStruct((T, D), jnp.float32),
        mesh=_sc_mesh,
        scratch_types=[
            pltpu.VMEM((TPW,), jnp.int32),
            pltpu.VMEM((TPW,), jnp.int32),
            pltpu.VMEM((TPW, L), jnp.float32),
            pltpu.VMEM((TPW, L), jnp.float32),
            pltpu.VMEM((L, D), jnp.float32),
            pltpu.VMEM((L, D), jnp.float32),
            pltpu.VMEM((L, D), jnp.float32),
            pltpu.SemaphoreType.DMA,
        ],
    )
    return f(y, dst0, dst1, w0, w1)


def kernel(inputs, gate_W, gate_b, W1, b1, W2, b2):
    x = inputs.reshape(T, D)
    ef, rf, w, counts, ends8, offs8 = _gate_call(x, gate_W, gate_b)
    offsb = jnp.broadcast_to(offs8.reshape(E, 1), (E, L))

    e0 = ef[:, 0].astype(jnp.int32)
    e1 = ef[:, 1].astype(jnp.int32)
    r0 = rf[:, 0].astype(jnp.int32)
    r1 = rf[:, 1].astype(jnp.int32)

    dst0, dst1, xd = _dispatch_sc(offsb, e0, e1, r0, r1, x)
    te = jnp.clip(
        jnp.searchsorted(ends8.reshape(E), jnp.arange(NT, dtype=jnp.int32) * TM,
                         side="right"),
        0, E - 1).astype(jnp.int32)

    chg = jnp.concatenate([jnp.ones((1,), jnp.int32),
                           (te[1:] != te[:-1]).astype(jnp.int32)])
    visit = jnp.cumsum(chg) - 1
    slot = (visit % 2).astype(jnp.int32)
    ev = jnp.zeros((NT,), jnp.int32).at[visit].set(te)
    nvisit = visit[-1] + 1
    hn = (visit < nvisit - 1).astype(jnp.int32)
    nxt = ev[jnp.minimum(visit + 1, NT - 1)].astype(jnp.int32)

    y = _ffn_call(te, chg, slot, nxt, hn, xd, W1, b1, W2, b2)

    out = _combine_sc(y, dst0, dst1,
                      jnp.broadcast_to(w[:, 0:1], (T, L)),
                      jnp.broadcast_to(w[:, 1:2], (T, L)))
    return out.reshape(B, S, D)
